# baseline (speedup 1.0000x reference)
import functools

import jax
import jax.numpy as jnp
from jax import lax
from jax.experimental import pallas as pl
from jax.experimental.pallas import tpu as pltpu
from jax.experimental.pallas import tpu_sc as plsc

B, T, H, C = 8, 2048, 128, 32
ENC = 1024
TM = T - ENC

NC, NS = 2, 16
NW = NC * NS
ROWS_PER_W = (B * TM) // NW
CHUNK = 128
NCHUNK = ROWS_PER_W // CHUNK
W_PER_B = TM // ROWS_PER_W


def _sc_gather_body(shuffle_hbm, spikes_hbm, out_hbm, idx_v, rows_v, sem):
    wid = lax.axis_index("s") * NC + lax.axis_index("c")
    b = wid // W_PER_B
    t_base = (wid % W_PER_B) * ROWS_PER_W
    for j in range(NCHUNK):
        pltpu.sync_copy(shuffle_hbm.at[pl.ds(ENC + t_base + j * CHUNK, CHUNK)],
                        idx_v.at[j])
    cps = [
        pltpu.async_copy(spikes_hbm.at[b].at[idx_v.at[j]],
                         rows_v.at[pl.ds(j * CHUNK, CHUNK)], sem)
        for j in range(NCHUNK)
    ]
    for cp in cps:
        cp.wait()
    pltpu.sync_copy(rows_v, out_hbm.at[pl.ds(wid * ROWS_PER_W, ROWS_PER_W)])


_sc_gather = functools.partial(
    pl.kernel,
    mesh=plsc.VectorSubcoreMesh(core_axis_name="c", subcore_axis_name="s"),
    out_type=jax.ShapeDtypeStruct((B * TM, C), jnp.int32),
    scratch_types=[
        pltpu.VMEM((NCHUNK, CHUNK), jnp.int32),
        pltpu.VMEM((ROWS_PER_W, C), jnp.int32),
        pltpu.SemaphoreType.DMA,
    ],
    compiler_params=pltpu.CompilerParams(use_tc_tiling_on_sc=False),
)(_sc_gather_body)


def _tc_loss_body(lengths_ref, tokpos_ref, bf_ref, tgt_ref,
                  w1_ref, b1_ref, w2_ref, b2_ref, out_ref, acc_ref):
    b = pl.program_id(0)
    x = bf_ref[0]
    h = jnp.dot(x, w1_ref[...], preferred_element_type=jnp.float32) + b1_ref[...]
    h = jax.nn.gelu(h)
    lr = jnp.dot(h, w2_ref[...], preferred_element_type=jnp.float32) + b2_ref[...]
    tgt = tgt_ref[...].astype(jnp.float32)
    loss = jnp.exp(lr) - tgt * lr
    mask = tokpos_ref[...] < lengths_ref[b]
    loss = jnp.where(mask, loss, 0.0)

    @pl.when(b == 0)
    def _():
        acc_ref[0] = 0.0
        acc_ref[1] = 0.0

    acc_ref[0] += jnp.sum(loss)
    acc_ref[1] += jnp.sum(mask.astype(jnp.float32))

    @pl.when(b == B - 1)
    def _():
        denom = jnp.maximum(acc_ref[1] * C, 1.0)
        out_ref[0, 0] = acc_ref[0] / denom


_tc_loss = pl.pallas_call(
    _tc_loss_body,
    grid=(B,),
    in_specs=[
        pl.BlockSpec(memory_space=pltpu.SMEM),
        pl.BlockSpec((TM, 1), lambda b: (0, 0)),
        pl.BlockSpec((1, TM, H), lambda b: (b, 0, 0)),
        pl.BlockSpec((TM, C), lambda b: (b, 0)),
        pl.BlockSpec((H, H), lambda b: (0, 0)),
        pl.BlockSpec((1, H), lambda b: (0, 0)),
        pl.BlockSpec((H, C), lambda b: (0, 0)),
        pl.BlockSpec((1, C), lambda b: (0, 0)),
    ],
    out_specs=pl.BlockSpec(memory_space=pltpu.SMEM),
    out_shape=jax.ShapeDtypeStruct((1, 1), jnp.float32),
    scratch_shapes=[pltpu.SMEM((2,), jnp.float32)],
)


def kernel(backbone_features, spikes, shuffle, lengths, encoder_frac, W1, b1, W2, b2):
    del encoder_frac
    target = _sc_gather(shuffle, spikes)
    tokpos = shuffle[ENC:].reshape(TM, 1)
    out = _tc_loss(lengths, tokpos, backbone_features, target,
                   W1, b1.reshape(1, H), W2, b2.reshape(1, C))
    return out[0, 0]


# speedup vs baseline: 1.0069x; 1.0069x over previous
import functools

import jax
import jax.numpy as jnp
from jax import lax
from jax.experimental import pallas as pl
from jax.experimental.pallas import tpu as pltpu
from jax.experimental.pallas import tpu_sc as plsc

B, T, H, C = 8, 2048, 128, 32
ENC = 1024
TM = T - ENC

NC, NS = 2, 16
NW = NC * NS
ROWS_PER_W = (B * TM) // NW
CHUNK = 128
NCHUNK = ROWS_PER_W // CHUNK
W_PER_B = TM // ROWS_PER_W


PLANE = C * T // 128


def _sc_gather_body(shuffle_hbm, spikes_hbm, out_hbm, idx_v, plane_v, rows_v, sem):
    wid = lax.axis_index("s") * NC + lax.axis_index("c")
    b = wid // W_PER_B
    t_base = (wid % W_PER_B) * ROWS_PER_W
    cp = pltpu.async_copy(spikes_hbm.at[pl.ds(b * PLANE, PLANE)], plane_v, sem)
    for j in range(NCHUNK):
        pltpu.sync_copy(shuffle_hbm.at[pl.ds(ENC + t_base + j * CHUNK, CHUNK)],
                        idx_v.at[j])
    cp.wait()
    lane = lax.iota(jnp.int32, 16)
    for g in range(ROWS_PER_W // 16):
        t16 = idx_v[g // 8, pl.ds((g % 8) * 16, 16)]
        row_base = lax.shift_right_logical(t16, 7) * 8
        col = lax.bitwise_and(t16, 127)
        lrow = g * 16 + lane
        for c in range(C):
            r16 = row_base + ((c // 8) * (T // 128) * 8 + (c % 8))
            v16 = plsc.load_gather(plane_v, [r16, col])
            plsc.store_scatter(rows_v, [lrow, jnp.full((16,), c, jnp.int32)], v16)
    pltpu.sync_copy(rows_v, out_hbm.at[pl.ds(wid * ROWS_PER_W, ROWS_PER_W)])


_sc_gather = functools.partial(
    pl.kernel,
    mesh=plsc.VectorSubcoreMesh(core_axis_name="c", subcore_axis_name="s"),
    out_type=jax.ShapeDtypeStruct((B * TM, C), jnp.int32),
    scratch_types=[
        pltpu.VMEM((NCHUNK, CHUNK), jnp.int32),
        pltpu.VMEM((PLANE, 128), jnp.int32),
        pltpu.VMEM((ROWS_PER_W, C), jnp.int32),
        pltpu.SemaphoreType.DMA,
    ],
    compiler_params=pltpu.CompilerParams(use_tc_tiling_on_sc=False,
                                         needs_layout_passes=False),
)(_sc_gather_body)


def _tc_loss_body(lengths_ref, tokpos_ref, bf_ref, tgt_ref,
                  w1_ref, b1_ref, w2_ref, b2_ref, out_ref, acc_ref):
    b = pl.program_id(0)
    x = bf_ref[0]
    h = jnp.dot(x, w1_ref[...], preferred_element_type=jnp.float32) + b1_ref[...]
    h = jax.nn.gelu(h)
    lr = jnp.dot(h, w2_ref[...], preferred_element_type=jnp.float32) + b2_ref[...]
    tgt = tgt_ref[...].astype(jnp.float32)
    loss = jnp.exp(lr) - tgt * lr
    mask = tokpos_ref[...] < lengths_ref[b]
    loss = jnp.where(mask, loss, 0.0)

    @pl.when(b == 0)
    def _():
        acc_ref[0] = 0.0
        acc_ref[1] = 0.0

    acc_ref[0] += jnp.sum(loss)
    acc_ref[1] += jnp.sum(mask.astype(jnp.float32))

    @pl.when(b == B - 1)
    def _():
        denom = jnp.maximum(acc_ref[1] * C, 1.0)
        out_ref[0, 0] = acc_ref[0] / denom


_tc_loss = pl.pallas_call(
    _tc_loss_body,
    grid=(B,),
    in_specs=[
        pl.BlockSpec(memory_space=pltpu.SMEM),
        pl.BlockSpec((TM, 1), lambda b: (0, 0)),
        pl.BlockSpec((1, TM, H), lambda b: (b, 0, 0)),
        pl.BlockSpec((TM, C), lambda b: (b, 0)),
        pl.BlockSpec((H, H), lambda b: (0, 0)),
        pl.BlockSpec((1, H), lambda b: (0, 0)),
        pl.BlockSpec((H, C), lambda b: (0, 0)),
        pl.BlockSpec((1, C), lambda b: (0, 0)),
    ],
    out_specs=pl.BlockSpec(memory_space=pltpu.SMEM),
    out_shape=jax.ShapeDtypeStruct((1, 1), jnp.float32),
    scratch_shapes=[pltpu.SMEM((2,), jnp.float32)],
)


def kernel(backbone_features, spikes, shuffle, lengths, encoder_frac, W1, b1, W2, b2):
    del encoder_frac
    spikes_view = (spikes.reshape(B, T // 128, 128, C // 8, 8)
                   .transpose(0, 3, 1, 4, 2)
                   .reshape(B * PLANE, 128))
    target = _sc_gather(shuffle, spikes_view)
    tokpos = shuffle[ENC:].reshape(TM, 1)
    out = _tc_loss(lengths, tokpos, backbone_features, target,
                   W1, b1.reshape(1, H), W2, b2.reshape(1, C))
    return out[0, 0]


# speedup vs baseline: 1.1878x; 1.1797x over previous
import functools

import jax
import jax.numpy as jnp
from jax import lax
from jax.experimental import pallas as pl
from jax.experimental.pallas import tpu as pltpu
from jax.experimental.pallas import tpu_sc as plsc

B, T, H, C = 8, 2048, 128, 32
ENC = 1024
TM = T - ENC
LW = 128

NC, NS = 2, 16
NW = NC * NS
ROWS_PER_W = (B * TM) // NW
CHUNK = 128
NCHUNK = ROWS_PER_W // CHUNK
W_PER_B = TM // ROWS_PER_W


def _tc_transpose_body(spk_ref, out_ref):
    x = spk_ref[0].astype(jnp.float32)
    out_ref[:, :C] = jnp.transpose(x, (1, 0))


_tc_transpose = pl.pallas_call(
    _tc_transpose_body,
    grid=(B,),
    in_specs=[pl.BlockSpec((1, C, T), lambda b: (b, 0, 0))],
    out_specs=pl.BlockSpec((T, LW), lambda b: (b, 0)),
    out_shape=jax.ShapeDtypeStruct((B * T, LW), jnp.float32),
)


def _sc_gather_body(shuffle_hbm, rows_hbm, out_hbm, idx_v, rows_v, sem):
    wid = lax.axis_index("s") * NC + lax.axis_index("c")
    b = wid // W_PER_B
    t_base = (wid % W_PER_B) * ROWS_PER_W
    for j in range(NCHUNK):
        pltpu.sync_copy(shuffle_hbm.at[pl.ds(ENC + t_base + j * CHUNK, CHUNK)],
                        idx_v.at[j])
    off = b * T
    for j in range(NCHUNK):
        for i in range(CHUNK // 16):
            sl = (j, pl.ds(i * 16, 16))
            idx_v[sl] = idx_v[sl] + off
    cps = [
        pltpu.async_copy(rows_hbm.at[idx_v.at[j]],
                         rows_v.at[pl.ds(j * CHUNK, CHUNK)], sem)
        for j in range(NCHUNK)
    ]
    for cp in cps:
        cp.wait()
    pltpu.sync_copy(rows_v, out_hbm.at[pl.ds(wid * ROWS_PER_W, ROWS_PER_W)])


_sc_gather = functools.partial(
    pl.kernel,
    mesh=plsc.VectorSubcoreMesh(core_axis_name="c", subcore_axis_name="s"),
    out_type=jax.ShapeDtypeStruct((B * TM, LW), jnp.float32),
    scratch_types=[
        pltpu.VMEM((NCHUNK, CHUNK), jnp.int32),
        pltpu.VMEM((ROWS_PER_W, LW), jnp.float32),
        pltpu.SemaphoreType.DMA,
    ],
    compiler_params=pltpu.CompilerParams(use_tc_tiling_on_sc=False),
)(_sc_gather_body)


def _tc_loss_body(lengths_ref, tokpos_ref, bf_ref, tgt_ref,
                  w1_ref, b1_ref, w2_ref, b2_ref, out_ref, acc_ref):
    b = pl.program_id(0)
    x = bf_ref[0]
    h = jnp.dot(x, w1_ref[...], preferred_element_type=jnp.float32) + b1_ref[...]
    h = jax.nn.gelu(h)
    lr = jnp.dot(h, w2_ref[...], preferred_element_type=jnp.float32) + b2_ref[...]
    tgt = tgt_ref[:, :C]
    loss = jnp.exp(lr) - tgt * lr
    mask = tokpos_ref[...] < lengths_ref[b]
    loss = jnp.where(mask, loss, 0.0)

    @pl.when(b == 0)
    def _():
        acc_ref[0] = 0.0
        acc_ref[1] = 0.0

    acc_ref[0] += jnp.sum(loss)
    acc_ref[1] += jnp.sum(mask.astype(jnp.float32))

    @pl.when(b == B - 1)
    def _():
        denom = jnp.maximum(acc_ref[1] * C, 1.0)
        out_ref[0, 0] = acc_ref[0] / denom


_tc_loss = pl.pallas_call(
    _tc_loss_body,
    grid=(B,),
    in_specs=[
        pl.BlockSpec(memory_space=pltpu.SMEM),
        pl.BlockSpec((TM, 1), lambda b: (0, 0)),
        pl.BlockSpec((1, TM, H), lambda b: (b, 0, 0)),
        pl.BlockSpec((TM, LW), lambda b: (b, 0)),
        pl.BlockSpec((H, H), lambda b: (0, 0)),
        pl.BlockSpec((1, H), lambda b: (0, 0)),
        pl.BlockSpec((H, C), lambda b: (0, 0)),
        pl.BlockSpec((1, C), lambda b: (0, 0)),
    ],
    out_specs=pl.BlockSpec(memory_space=pltpu.SMEM),
    out_shape=jax.ShapeDtypeStruct((1, 1), jnp.float32),
    scratch_shapes=[pltpu.SMEM((2,), jnp.float32)],
)


def kernel(backbone_features, spikes, shuffle, lengths, encoder_frac, W1, b1, W2, b2):
    del encoder_frac
    spikes_t = jnp.swapaxes(spikes, 1, 2)
    rows = _tc_transpose(spikes_t)
    target = _sc_gather(shuffle, rows)
    tokpos = shuffle[ENC:].reshape(TM, 1)
    out = _tc_loss(lengths, tokpos, backbone_features, target,
                   W1, b1.reshape(1, H), W2, b2.reshape(1, C))
    return out[0, 0]
